# baseline (speedup 1.0000x reference)
import functools

import jax
import jax.numpy as jnp
from jax import lax
from jax.experimental import pallas as pl
from jax.experimental.pallas import tpu as pltpu
from jax.experimental.pallas import tpu_sc as plsc

BATCH = 16384
EMBED_DIM = 32
NUM_CORES = 2
NUM_SUBCORES = 16
NUM_WORKERS = NUM_CORES * NUM_SUBCORES
ROWS_PER_WORKER = BATCH // NUM_WORKERS
CHUNK = 128
NUM_CHUNKS = ROWS_PER_WORKER // CHUNK


def _gather_sc(uidx, iidx, emb_user, emb_item):
    mesh = plsc.VectorSubcoreMesh(core_axis_name="c", subcore_axis_name="s")

    @functools.partial(
        pl.kernel,
        out_type=[
            jax.ShapeDtypeStruct((BATCH, EMBED_DIM), jnp.float32),
            jax.ShapeDtypeStruct((BATCH, EMBED_DIM), jnp.float32),
        ],
        mesh=mesh,
        compiler_params=pltpu.CompilerParams(use_tc_tiling_on_sc=False),
        scratch_types=[
            pltpu.VMEM((NUM_CHUNKS, CHUNK), jnp.int32),
            pltpu.VMEM((NUM_CHUNKS, CHUNK), jnp.int32),
            pltpu.VMEM((ROWS_PER_WORKER, EMBED_DIM), jnp.float32),
            pltpu.VMEM((ROWS_PER_WORKER, EMBED_DIM), jnp.float32),
            pltpu.SemaphoreType.DMA,
        ],
    )
    def gather_kernel(uidx_hbm, iidx_hbm, ut_hbm, it_hbm, ue_hbm, ie_hbm,
                      uidx_v, iidx_v, ur_v, ir_v, sem):
        wid = lax.axis_index("s") * NUM_CORES + lax.axis_index("c")
        base = wid * ROWS_PER_WORKER
        pltpu.sync_copy(uidx_hbm.at[wid], uidx_v)
        pltpu.sync_copy(iidx_hbm.at[wid], iidx_v)
        copies = []
        for j in range(NUM_CHUNKS):
            copies.append(pltpu.async_copy(
                ut_hbm.at[uidx_v.at[j]],
                ur_v.at[pl.ds(j * CHUNK, CHUNK)], sem))
            copies.append(pltpu.async_copy(
                it_hbm.at[iidx_v.at[j]],
                ir_v.at[pl.ds(j * CHUNK, CHUNK)], sem))
        for c in copies:
            c.wait()
        pltpu.sync_copy(ur_v, ue_hbm.at[pl.ds(base, ROWS_PER_WORKER)])
        pltpu.sync_copy(ir_v, ie_hbm.at[pl.ds(base, ROWS_PER_WORKER)])

    return gather_kernel(uidx, iidx, emb_user, emb_item)


def _mlp_body(ue_ref, ie_ref, w1_ref, b1_ref, w2_ref, b2_ref, w3_ref, b3_ref,
              w4_ref, b4_ref, o_ref):
    h = (jnp.dot(ue_ref[...], w1_ref[0:EMBED_DIM, :],
                 preferred_element_type=jnp.float32)
         + jnp.dot(ie_ref[...], w1_ref[EMBED_DIM:2 * EMBED_DIM, :],
                   preferred_element_type=jnp.float32)
         + b1_ref[...])
    h = jnp.maximum(h, 0.0)
    h = jnp.maximum(jnp.dot(h, w2_ref[...], preferred_element_type=jnp.float32)
                    + b2_ref[...], 0.0)
    h = jnp.maximum(jnp.dot(h, w3_ref[...], preferred_element_type=jnp.float32)
                    + b3_ref[...], 0.0)
    y = jax.nn.sigmoid(jnp.dot(h, w4_ref[...], preferred_element_type=jnp.float32)
                       + b4_ref[...])
    o_ref[...] = y * 5.0 + 1.0


def _mlp_tc(ue, ie, W1, b1, W2, b2, W3, b3, W4, b4):
    blk = 2048
    grid = (BATCH // blk,)
    full = lambda shape: pl.BlockSpec(shape, lambda i: (0, 0))
    return pl.pallas_call(
        _mlp_body,
        grid=grid,
        in_specs=[
            pl.BlockSpec((blk, EMBED_DIM), lambda i: (i, 0)),
            pl.BlockSpec((blk, EMBED_DIM), lambda i: (i, 0)),
            full(W1.shape), full(b1.shape),
            full(W2.shape), full(b2.shape),
            full(W3.shape), full(b3.shape),
            full(W4.shape), full(b4.shape),
        ],
        out_specs=pl.BlockSpec((blk, 1), lambda i: (i, 0)),
        out_shape=jax.ShapeDtypeStruct((BATCH, 1), jnp.float32),
    )(ue, ie, W1, b1, W2, b2, W3, b3, W4, b4)


def kernel(user_indices, item_indices, emb_user, emb_item,
           W1, b1, W2, b2, W3, b3, W4, b4):
    uidx = user_indices.astype(jnp.int32).reshape(NUM_WORKERS, NUM_CHUNKS, CHUNK)
    iidx = item_indices.astype(jnp.int32).reshape(NUM_WORKERS, NUM_CHUNKS, CHUNK)
    ue, ie = _gather_sc(uidx, iidx, emb_user, emb_item)
    return _mlp_tc(ue, ie, W1, b1.reshape(1, -1), W2, b2.reshape(1, -1),
                   W3, b3.reshape(1, -1), W4, b4.reshape(1, -1))
